# Initial kernel scaffold; baseline (speedup 1.0000x reference)
import jax
import jax.numpy as jnp
from jax.experimental import pallas as pl


def kernel(V_set, node_ids):
    raise NotImplementedError("write your pallas kernel here")


# speedup vs baseline: 3.5313x; 3.5313x over previous
import functools

import jax
import jax.numpy as jnp
from jax import lax
from jax.experimental import pallas as pl
from jax.experimental.pallas import tpu as pltpu
from jax.experimental.pallas import tpu_sc as plsc

N_NODES = 10000
D_FEAT = 128
N_EDGES = 320000

NC = 2
NS = 16
NW = NC * NS

B_W = N_EDGES // NW
CHUNK = 80
N_CHUNK = B_W // CHUNK


def _make_gather():
    mesh = plsc.VectorSubcoreMesh(
        core_axis_name="c", subcore_axis_name="s", num_cores=NC, num_subcores=NS
    )

    @functools.partial(
        pl.kernel,
        out_type=jax.ShapeDtypeStruct((N_EDGES, D_FEAT), jnp.float32),
        mesh=mesh,
        scratch_types=[
            pltpu.VMEM((N_CHUNK, CHUNK), jnp.int32),
            pltpu.VMEM((CHUNK, D_FEAT), jnp.float32),
            pltpu.SemaphoreType.DMA,
        ],
    )
    def gather_kernel(table_hbm, idx_hbm, out_hbm, idx_v, rows_v, sem):
        wid = lax.axis_index("s") * NC + lax.axis_index("c")
        base = wid * B_W
        pltpu.sync_copy(idx_hbm.at[wid], idx_v)

        @pl.loop(0, N_CHUNK)
        def _chunk(j):
            pltpu.async_copy(table_hbm.at[idx_v.at[j]], rows_v, sem).wait()
            pltpu.sync_copy(rows_v, out_hbm.at[pl.ds(base + j * CHUNK, CHUNK)])

    return gather_kernel


_gather = _make_gather()


@jax.jit
def kernel(V_set, node_ids):
    table = V_set[0]
    idx = node_ids.reshape(NW, N_CHUNK, CHUNK)
    out = _gather(table, idx)
    return out[jnp.newaxis]


# speedup vs baseline: 5.7698x; 1.6339x over previous
import functools

import jax
import jax.numpy as jnp
from jax import lax
from jax.experimental import pallas as pl
from jax.experimental.pallas import tpu as pltpu
from jax.experimental.pallas import tpu_sc as plsc

N_NODES = 10000
D_FEAT = 128
N_EDGES = 320000

NC = 2
NS = 16
NW = NC * NS

B_W = N_EDGES // NW
CHUNK = 80
N_CHUNK = B_W // CHUNK
NBUF = 5


def _make_gather():
    mesh = plsc.VectorSubcoreMesh(
        core_axis_name="c", subcore_axis_name="s", num_cores=NC, num_subcores=NS
    )

    @functools.partial(
        pl.kernel,
        out_type=jax.ShapeDtypeStruct((N_EDGES, D_FEAT), jnp.float32),
        mesh=mesh,
        scratch_types=[
            pltpu.VMEM((N_CHUNK, CHUNK), jnp.int32),
            pltpu.VMEM((NBUF, CHUNK, D_FEAT), jnp.float32),
            pltpu.SemaphoreType.DMA((NBUF,)),
            pltpu.SemaphoreType.DMA((NBUF,)),
        ],
    )
    def gather_kernel(table_hbm, idx_hbm, out_hbm, idx_v, rows_v, gsem, ssem):
        wid = lax.axis_index("s") * NC + lax.axis_index("c")
        base = wid * B_W
        pltpu.sync_copy(idx_hbm.at[wid], idx_v)

        def start_gather(j, b):
            pltpu.async_copy(table_hbm.at[idx_v.at[j]], rows_v.at[b], gsem.at[b])

        def wait_gather(j, b):
            pltpu.make_async_copy(
                table_hbm.at[idx_v.at[j]], rows_v.at[b], gsem.at[b]
            ).wait()

        def store_slot(j):
            return out_hbm.at[pl.ds(base + j * CHUNK, CHUNK)]

        def start_store(j, b):
            pltpu.async_copy(rows_v.at[b], store_slot(j), ssem.at[b])

        def wait_store(j, b):
            pltpu.make_async_copy(rows_v.at[b], store_slot(j), ssem.at[b]).wait()

        for b in range(NBUF):
            start_gather(b, b)

        @pl.loop(0, N_CHUNK - NBUF, step=NBUF)
        def _ring(j0):
            for b in range(NBUF):
                j = j0 + b
                wait_gather(j, b)
                start_store(j, b)
                wait_store(j, b)
                start_gather(j + NBUF, b)

        for b in range(NBUF):
            j = N_CHUNK - NBUF + b
            wait_gather(j, b)
            start_store(j, b)
            wait_store(j, b)

    return gather_kernel


_gather = _make_gather()


@jax.jit
def kernel(V_set, node_ids):
    table = V_set[0]
    idx = node_ids.reshape(NW, N_CHUNK, CHUNK)
    out = _gather(table, idx)
    return out[jnp.newaxis]
